# Initial kernel scaffold; baseline (speedup 1.0000x reference)
import jax
import jax.numpy as jnp
from jax.experimental import pallas as pl


def kernel(edge_index, user_emb, item_emb):
    raise NotImplementedError("write your pallas kernel here")


# speedup vs baseline: 7.6391x; 7.6391x over previous
import functools

import jax
import jax.numpy as jnp
from jax import lax
from jax.experimental import pallas as pl
from jax.experimental.pallas import tpu as pltpu
from jax.experimental.pallas import tpu_sc as plsc

N_USERS = 50000
N_NODES = 100000
DIM = 32
N_LAYERS = 3

NC = 2
NS = 16
LANES = 16
CHUNK = 128
HALF = N_NODES // NC
ACC_ROWS = HALF + LANES

_MESH = plsc.VectorSubcoreMesh(core_axis_name="c", subcore_axis_name="s")
_SC_PARAMS = pltpu.CompilerParams(use_tc_tiling_on_sc=False)


def _remap_col(col_v, lcol_v, base):
    for j in range(CHUNK // LANES):
        x = col_v[pl.ds(j * LANES, LANES)]
        lc = x - base
        ok = (lc >= 0) & (lc < HALF)
        lcol_v[pl.ds(j * LANES, LANES)] = jnp.where(ok, lc, HALF)


STRIPE = 3200
LAST_STRIPE = HALF - (NS - 1) * STRIPE
BOUNCE = 400


def _sc_scatter_rows(src, row_p, col_p):
    e_pad = row_p.shape[0]
    per_tile = e_pad // NS
    n_chunks = per_tile // CHUNK

    @functools.partial(
        pl.kernel,
        out_type=jax.ShapeDtypeStruct((N_NODES, DIM), jnp.float32),
        mesh=_MESH,
        scratch_types=[
            pltpu.VMEM((CHUNK,), jnp.int32),
            pltpu.VMEM((CHUNK,), jnp.int32),
            pltpu.VMEM((CHUNK,), jnp.int32),
            pltpu.VMEM((CHUNK, DIM), jnp.float32),
            pltpu.VMEM((BOUNCE, DIM), jnp.float32),
            pltpu.VMEM_SHARED((ACC_ROWS, DIM), jnp.float32),
            pltpu.SemaphoreType.DMA,
        ],
        compiler_params=_SC_PARAMS,
    )
    def k(src_hbm, row_hbm, col_hbm, out_hbm,
          row_v, col_v, lcol_v, rows_v, bounce, acc, sem):
        core = lax.axis_index("c")
        tile = lax.axis_index("s")
        base = core * HALF
        tile_e0 = tile * per_tile
        r0 = tile * STRIPE

        def zfill(i, carry):
            bounce[i, pl.ds(0, LANES)] = jnp.zeros((LANES,), jnp.float32)
            bounce[i, pl.ds(LANES, LANES)] = jnp.zeros((LANES,), jnp.float32)
            return carry
        lax.fori_loop(0, BOUNCE, zfill, 0)

        n_b = jnp.where(tile < NS - 1, STRIPE // BOUNCE, LAST_STRIPE // BOUNCE)

        def zcopy(i, carry):
            pltpu.sync_copy(bounce, acc.at[pl.ds(r0 + i * BOUNCE, BOUNCE)])
            return carry
        lax.fori_loop(0, n_b, zcopy, 0)
        plsc.subcore_barrier()

        def body(i, carry):
            e0 = pl.multiple_of(tile_e0 + i * CHUNK, CHUNK)
            pltpu.sync_copy(row_hbm.at[pl.ds(e0, CHUNK)], row_v)
            pltpu.sync_copy(col_hbm.at[pl.ds(e0, CHUNK)], col_v)
            pltpu.async_copy(src_hbm.at[row_v], rows_v, sem).wait()
            _remap_col(col_v, lcol_v, base)
            pltpu.sync_copy(rows_v, acc.at[lcol_v], add=True)
            return carry

        lax.fori_loop(0, n_chunks, body, 0)
        plsc.subcore_barrier()

        def dcopy(i, carry):
            o = r0 + i * BOUNCE
            pltpu.sync_copy(acc.at[pl.ds(o, BOUNCE)], bounce)
            pltpu.sync_copy(bounce, out_hbm.at[pl.ds(base + o, BOUNCE)])
            return carry
        lax.fori_loop(0, n_b, dcopy, 0)

    return k(src, row_p, col_p)


def _sc_degree(col_p):
    e_pad = col_p.shape[0]
    per_tile = e_pad // NS
    n_chunks = per_tile // CHUNK

    @functools.partial(
        pl.kernel,
        out_type=jax.ShapeDtypeStruct((N_NODES,), jnp.float32),
        mesh=_MESH,
        scratch_types=[
            pltpu.VMEM((CHUNK,), jnp.int32),
            pltpu.VMEM((CHUNK,), jnp.int32),
            pltpu.VMEM((CHUNK,), jnp.float32),
            pltpu.VMEM((STRIPE,), jnp.float32),
            pltpu.VMEM_SHARED((ACC_ROWS,), jnp.float32),
            pltpu.SemaphoreType.DMA,
        ],
        compiler_params=_SC_PARAMS,
    )
    def k(col_hbm, deg_hbm, col_v, lcol_v, ones_v, bounce, acc, sem):
        core = lax.axis_index("c")
        tile = lax.axis_index("s")
        base = core * HALF
        tile_e0 = tile * per_tile
        r0 = tile * STRIPE

        for j in range(CHUNK // LANES):
            ones_v[pl.ds(j * LANES, LANES)] = jnp.ones((LANES,), jnp.float32)

        def zfill(i, carry):
            o = pl.multiple_of(i * LANES, LANES)
            bounce[pl.ds(o, LANES)] = jnp.zeros((LANES,), jnp.float32)
            return carry
        lax.fori_loop(0, STRIPE // LANES, zfill, 0)

        @pl.when(tile < NS - 1)
        def _():
            pltpu.sync_copy(bounce, acc.at[pl.ds(r0, STRIPE)])

        @pl.when(tile == NS - 1)
        def _():
            pltpu.sync_copy(bounce.at[pl.ds(0, LAST_STRIPE)],
                            acc.at[pl.ds(r0, LAST_STRIPE)])
        plsc.subcore_barrier()

        def body(i, carry):
            e0 = pl.multiple_of(tile_e0 + i * CHUNK, CHUNK)
            pltpu.sync_copy(col_hbm.at[pl.ds(e0, CHUNK)], col_v)
            _remap_col(col_v, lcol_v, base)
            pltpu.sync_copy(ones_v, acc.at[lcol_v], add=True)
            return carry

        lax.fori_loop(0, n_chunks, body, 0)
        plsc.subcore_barrier()

        @pl.when(tile < NS - 1)
        def _():
            pltpu.sync_copy(acc.at[pl.ds(r0, STRIPE)], bounce)
            pltpu.sync_copy(bounce, deg_hbm.at[pl.ds(base + r0, STRIPE)])

        @pl.when(tile == NS - 1)
        def _():
            pltpu.sync_copy(acc.at[pl.ds(r0, LAST_STRIPE)],
                            bounce.at[pl.ds(0, LAST_STRIPE)])
            pltpu.sync_copy(bounce.at[pl.ds(0, LAST_STRIPE)],
                            deg_hbm.at[pl.ds(base + r0, LAST_STRIPE)])

    return k(col_p)


_BLK = 4000


def _tc_prep(deg2, emb):
    def body(deg_ref, emb_ref, dinv_ref, dinv2_ref, s_ref):
        d = deg_ref[...]
        di = jnp.where(d > 0, lax.rsqrt(d), 0.0)
        dib = jnp.broadcast_to(di, (_BLK, DIM))
        dinv_ref[...] = dib
        dinv2_ref[...] = dib * dib
        s_ref[...] = dib * emb_ref[...]

    sh = jax.ShapeDtypeStruct((N_NODES, DIM), jnp.float32)
    return pl.pallas_call(
        body,
        grid=(N_NODES // _BLK,),
        in_specs=[pl.BlockSpec((_BLK, 1), lambda i: (i, 0)),
                  pl.BlockSpec((_BLK, DIM), lambda i: (i, 0))],
        out_specs=[pl.BlockSpec((_BLK, DIM), lambda i: (i, 0))] * 3,
        out_shape=[sh, sh, sh],
    )(deg2, emb)


def _tc_scale(acc, dinv_b, dinv2_b, sum_in):
    def body(a_ref, d_ref, d2_ref, su_ref, s_ref, so_ref):
        a = a_ref[...]
        s_ref[...] = d2_ref[...] * a
        so_ref[...] = su_ref[...] + d_ref[...] * a

    sh = jax.ShapeDtypeStruct((N_NODES, DIM), jnp.float32)
    spec = pl.BlockSpec((_BLK, DIM), lambda i: (i, 0))
    return pl.pallas_call(
        body,
        grid=(N_NODES // _BLK,),
        in_specs=[spec] * 4,
        out_specs=[spec] * 2,
        out_shape=[sh, sh],
    )(acc, dinv_b, dinv2_b, sum_in)


def _tc_final(acc, dinv_b, sum_in):
    def body(a_ref, d_ref, su_ref, o_ref):
        o_ref[...] = (su_ref[...] + d_ref[...] * a_ref[...]) * (
            1.0 / (N_LAYERS + 1))

    spec = pl.BlockSpec((_BLK, DIM), lambda i: (i, 0))
    return pl.pallas_call(
        body,
        grid=(N_NODES // _BLK,),
        in_specs=[spec] * 3,
        out_specs=spec,
        out_shape=jax.ShapeDtypeStruct((N_NODES, DIM), jnp.float32),
    )(acc, dinv_b, sum_in)


def kernel(edge_index, user_emb, item_emb):
    all_emb = jnp.concatenate([user_emb, item_emb], axis=0)
    row = edge_index[0].astype(jnp.int32)
    col = edge_index[1].astype(jnp.int32)

    n_edges = row.shape[0]
    step = NS * CHUNK
    e_pad = ((n_edges + step - 1) // step) * step
    pad = e_pad - n_edges
    row_p = jnp.concatenate([row, jnp.zeros((pad,), jnp.int32)])
    col_p = jnp.concatenate([col, jnp.full((pad,), N_NODES, jnp.int32)])

    deg = _sc_degree(col_p)
    dinv_b, dinv2_b, s = _tc_prep(deg.reshape(N_NODES, 1), all_emb)

    acc = _sc_scatter_rows(s, row_p, col_p)
    s, run = _tc_scale(acc, dinv_b, dinv2_b, all_emb)
    acc = _sc_scatter_rows(s, row_p, col_p)
    s, run = _tc_scale(acc, dinv_b, dinv2_b, run)
    acc = _sc_scatter_rows(s, row_p, col_p)
    final = _tc_final(acc, dinv_b, run)

    return (final[:N_USERS], final[N_USERS:])


# speedup vs baseline: 21.1855x; 2.7733x over previous
import functools

import jax
import jax.numpy as jnp
from jax import lax
from jax.experimental import pallas as pl
from jax.experimental.pallas import tpu as pltpu
from jax.experimental.pallas import tpu_sc as plsc

N_USERS = 50000
N_NODES = 100000
DIM = 32
HDIM = DIM // 2
N_LAYERS = 3

NC = 2
NS = 16
LANES = 16
CHUNK = 128
G = 8

ACC2_ROWS = N_NODES + 8
BOUNCE = 400
Z_STRIPE = 6400
Z_LAST = N_NODES - (NS - 1) * Z_STRIPE

_MESH = plsc.VectorSubcoreMesh(core_axis_name="c", subcore_axis_name="s")
_SC_PARAMS = pltpu.CompilerParams(use_tc_tiling_on_sc=False)


def _zero_stripes(acc, bounce, tile, width):
    def zfill(i, carry):
        for k in range(width // LANES):
            bounce[i, pl.ds(k * LANES, LANES)] = jnp.zeros((LANES,),
                                                           jnp.float32)
        return carry
    lax.fori_loop(0, BOUNCE, zfill, 0)

    n_b = jnp.where(tile < NS - 1, Z_STRIPE // BOUNCE, Z_LAST // BOUNCE)

    def zcopy(i, carry):
        pltpu.sync_copy(bounce,
                        acc.at[pl.ds(tile * Z_STRIPE + i * BOUNCE, BOUNCE)])
        return carry
    lax.fori_loop(0, n_b, zcopy, 0)
    return n_b


def _sc_scatter_rows(src2, row2, col2):
    total_chunks = row2.shape[0]
    per_tile = total_chunks // NS
    n_groups = per_tile // G

    @functools.partial(
        pl.kernel,
        out_type=jax.ShapeDtypeStruct((NC, N_NODES, HDIM), jnp.float32),
        mesh=_MESH,
        scratch_types=[
            pltpu.VMEM((G, CHUNK), jnp.int32),
            pltpu.VMEM((G, CHUNK), jnp.int32),
            pltpu.VMEM((G, CHUNK), jnp.int32),
            pltpu.VMEM((G, CHUNK, HDIM), jnp.float32),
            pltpu.VMEM((BOUNCE, HDIM), jnp.float32),
            pltpu.VMEM_SHARED((ACC2_ROWS, HDIM), jnp.float32),
            pltpu.SemaphoreType.DMA,
            pltpu.SemaphoreType.DMA,
        ],
        compiler_params=_SC_PARAMS,
    )
    def k(src_hbm, row_hbm, col_hbm, out_hbm,
          rowg, ridx, colg, bufs, bounce, acc, gsem, ssem):
        core = lax.axis_index("c")
        tile = lax.axis_index("s")
        c0_tile = tile * per_tile

        _zero_stripes(acc, bounce, tile, HDIM)
        plsc.subcore_barrier()

        def group(g, carry):
            c0 = pl.multiple_of(c0_tile + g * G, G)
            pltpu.sync_copy(row_hbm.at[pl.ds(c0, G)], rowg)
            pltpu.sync_copy(col_hbm.at[pl.ds(c0, G)], colg)
            for j in range(G):
                for kk in range(CHUNK // LANES):
                    o = pl.ds(kk * LANES, LANES)
                    ridx[j, o] = rowg[j, o] * 2 + core
            gd = [pltpu.async_copy(src_hbm.at[ridx.at[j]], bufs.at[j], gsem)
                  for j in range(G)]
            sd = []
            for j in range(G):
                gd[j].wait()
                sd.append(pltpu.async_copy(bufs.at[j], acc.at[colg.at[j]],
                                           ssem, add=True))
            for d in sd:
                d.wait()
            return carry

        lax.fori_loop(0, n_groups, group, 0)
        plsc.subcore_barrier()

        n_b = jnp.where(tile < NS - 1, Z_STRIPE // BOUNCE, Z_LAST // BOUNCE)

        def dcopy(i, carry):
            o = tile * Z_STRIPE + i * BOUNCE
            pltpu.sync_copy(acc.at[pl.ds(o, BOUNCE)], bounce)
            pltpu.sync_copy(bounce, out_hbm.at[core, pl.ds(o, BOUNCE)])
            return carry
        lax.fori_loop(0, n_b, dcopy, 0)

    return k(src2, row2, col2)


def _sc_degree(col2):
    total_chunks = col2.shape[0]
    per_core = total_chunks // NC
    per_tile = per_core // NS
    n_groups = per_tile // G

    @functools.partial(
        pl.kernel,
        out_type=jax.ShapeDtypeStruct((NC, N_NODES), jnp.float32),
        mesh=_MESH,
        scratch_types=[
            pltpu.VMEM((G, CHUNK), jnp.int32),
            pltpu.VMEM((CHUNK,), jnp.float32),
            pltpu.VMEM((Z_STRIPE,), jnp.float32),
            pltpu.VMEM_SHARED((ACC2_ROWS,), jnp.float32),
            pltpu.SemaphoreType.DMA,
        ],
        compiler_params=_SC_PARAMS,
    )
    def k(col_hbm, deg_hbm, colg, ones_v, bounce, acc, ssem):
        core = lax.axis_index("c")
        tile = lax.axis_index("s")
        c0_tile = (core * NS + tile) * per_tile

        for j in range(CHUNK // LANES):
            ones_v[pl.ds(j * LANES, LANES)] = jnp.ones((LANES,), jnp.float32)

        def zfill(i, carry):
            o = pl.multiple_of(i * LANES, LANES)
            bounce[pl.ds(o, LANES)] = jnp.zeros((LANES,), jnp.float32)
            return carry
        lax.fori_loop(0, Z_STRIPE // LANES, zfill, 0)

        @pl.when(tile < NS - 1)
        def _():
            pltpu.sync_copy(bounce, acc.at[pl.ds(tile * Z_STRIPE, Z_STRIPE)])

        @pl.when(tile == NS - 1)
        def _():
            pltpu.sync_copy(bounce.at[pl.ds(0, Z_LAST)],
                            acc.at[pl.ds(tile * Z_STRIPE, Z_LAST)])
        plsc.subcore_barrier()

        def group(g, carry):
            c0 = pl.multiple_of(c0_tile + g * G, G)
            pltpu.sync_copy(col_hbm.at[pl.ds(c0, G)], colg)
            sd = [pltpu.async_copy(ones_v, acc.at[colg.at[j]], ssem, add=True)
                  for j in range(G)]
            for d in sd:
                d.wait()
            return carry

        lax.fori_loop(0, n_groups, group, 0)
        plsc.subcore_barrier()

        @pl.when(tile < NS - 1)
        def _():
            o = tile * Z_STRIPE
            pltpu.sync_copy(acc.at[pl.ds(o, Z_STRIPE)], bounce)
            pltpu.sync_copy(bounce, deg_hbm.at[core, pl.ds(o, Z_STRIPE)])

        @pl.when(tile == NS - 1)
        def _():
            o = tile * Z_STRIPE
            pltpu.sync_copy(acc.at[pl.ds(o, Z_LAST)],
                            bounce.at[pl.ds(0, Z_LAST)])
            pltpu.sync_copy(bounce.at[pl.ds(0, Z_LAST)],
                            deg_hbm.at[core, pl.ds(o, Z_LAST)])

    return k(col2)


_BLK = 4000


def _tc_prep(deg_a, deg_b, emb):
    def body(da_ref, db_ref, emb_ref, dinv_ref, dinv2_ref, s_ref):
        d = da_ref[...] + db_ref[...]
        di = jnp.where(d > 0, lax.rsqrt(d), 0.0)
        dib = jnp.broadcast_to(di, (_BLK, DIM))
        dinv_ref[...] = dib
        dinv2_ref[...] = dib * dib
        s_ref[...] = dib * emb_ref[...]

    sh = jax.ShapeDtypeStruct((N_NODES, DIM), jnp.float32)
    dspec = pl.BlockSpec((_BLK, 1), lambda i: (i, 0))
    espec = pl.BlockSpec((_BLK, DIM), lambda i: (i, 0))
    return pl.pallas_call(
        body,
        grid=(N_NODES // _BLK,),
        in_specs=[dspec, dspec, espec],
        out_specs=[espec] * 3,
        out_shape=[sh, sh, sh],
    )(deg_a, deg_b, emb)


def _tc_scale(acc_lo, acc_hi, dinv_b, dinv2_b, sum_in):
    def body(al_ref, ah_ref, d_ref, d2_ref, su_ref, s_ref, so_ref):
        a = jnp.concatenate([al_ref[...], ah_ref[...]], axis=1)
        s_ref[...] = d2_ref[...] * a
        so_ref[...] = su_ref[...] + d_ref[...] * a

    sh = jax.ShapeDtypeStruct((N_NODES, DIM), jnp.float32)
    hspec = pl.BlockSpec((_BLK, HDIM), lambda i: (i, 0))
    spec = pl.BlockSpec((_BLK, DIM), lambda i: (i, 0))
    return pl.pallas_call(
        body,
        grid=(N_NODES // _BLK,),
        in_specs=[hspec, hspec, spec, spec, spec],
        out_specs=[spec] * 2,
        out_shape=[sh, sh],
    )(acc_lo, acc_hi, dinv_b, dinv2_b, sum_in)


def _tc_final(acc_lo, acc_hi, dinv_b, sum_in):
    def body(al_ref, ah_ref, d_ref, su_ref, o_ref):
        a = jnp.concatenate([al_ref[...], ah_ref[...]], axis=1)
        o_ref[...] = (su_ref[...] + d_ref[...] * a) * (1.0 / (N_LAYERS + 1))

    hspec = pl.BlockSpec((_BLK, HDIM), lambda i: (i, 0))
    spec = pl.BlockSpec((_BLK, DIM), lambda i: (i, 0))
    return pl.pallas_call(
        body,
        grid=(N_NODES // _BLK,),
        in_specs=[hspec, hspec, spec, spec],
        out_specs=spec,
        out_shape=jax.ShapeDtypeStruct((N_NODES, DIM), jnp.float32),
    )(acc_lo, acc_hi, dinv_b, sum_in)


def kernel(edge_index, user_emb, item_emb):
    all_emb = jnp.concatenate([user_emb, item_emb], axis=0)
    row = edge_index[0].astype(jnp.int32)
    col = edge_index[1].astype(jnp.int32)

    n_edges = row.shape[0]
    step = NC * NS * CHUNK * G
    e_pad = ((n_edges + step - 1) // step) * step
    pad = e_pad - n_edges
    row2 = jnp.concatenate([row, jnp.zeros((pad,), jnp.int32)])
    col2 = jnp.concatenate([col, jnp.full((pad,), N_NODES, jnp.int32)])
    row2 = row2.reshape(-1, CHUNK)
    col2 = col2.reshape(-1, CHUNK)

    deg = _sc_degree(col2)
    dinv_b, dinv2_b, s = _tc_prep(deg[0].reshape(N_NODES, 1),
                                  deg[1].reshape(N_NODES, 1), all_emb)

    def layer(s):
        acc = _sc_scatter_rows(s.reshape(2 * N_NODES, HDIM), row2, col2)
        return acc[0], acc[1]

    lo, hi = layer(s)
    s, run = _tc_scale(lo, hi, dinv_b, dinv2_b, all_emb)
    lo, hi = layer(s)
    s, run = _tc_scale(lo, hi, dinv_b, dinv2_b, run)
    lo, hi = layer(s)
    final = _tc_final(lo, hi, dinv_b, run)

    return (final[:N_USERS], final[N_USERS:])
